# baseline (speedup 1.0000x reference)
import functools

import jax
import jax.numpy as jnp
from jax.experimental import pallas as pl
from jax.experimental.pallas import tpu as pltpu

_PAIRS = 8
_NBUF = 3


def _body(nsteps, rows_per_pair, half, inp2_ref, inpr_ref, tab2_ref, out_ref,
          scratch, sems):
    i = pl.program_id(0)
    slot = jax.lax.rem(i, _NBUF)
    rows = rows_per_pair * _PAIRS

    @pl.when(i >= _NBUF)
    def _wait_prev():
        j = i - _NBUF
        pltpu.make_async_copy(
            scratch.at[slot], out_ref.at[pl.ds(j * rows, rows)], sems.at[slot]
        ).wait()

    for j in range(_PAIRS):
        m = i * _PAIRS + j
        base = j * rows_per_pair
        ab = inp2_ref[pl.ds(2 * m, 2)]
        scratch[slot, pl.ds(base, half)] = tab2_ref[pl.ds(0, half)] + ab[0:1]
        scratch[slot, pl.ds(base + half, 1)] = (
            tab2_ref[pl.ds(half, 1)] + inpr_ref[pl.ds(m, 1)])
        scratch[slot, pl.ds(base + half + 1, half)] = (
            tab2_ref[pl.ds(half + 1, half)] + ab[1:2])

    pltpu.make_async_copy(
        scratch.at[slot], out_ref.at[pl.ds(i * rows, rows)], sems.at[slot]
    ).start()

    @pl.when(i == nsteps - 1)
    def _drain():
        for k in range(_NBUF):
            j = nsteps - _NBUF + k
            s = j % _NBUF
            pltpu.make_async_copy(
                scratch.at[s], out_ref.at[pl.ds(j * rows, rows)], sems.at[s]
            ).wait()


def kernel(inputs, table):
    B, _, D = inputs.shape
    S1 = table.shape[0]
    lanes = 128
    total = B * S1 * D
    rows_per_pair = 2 * S1 * D // lanes
    half = (rows_per_pair - 1) // 2
    out_rows = total // lanes
    nsteps = B // (2 * _PAIRS)
    rows = rows_per_pair * _PAIRS

    tflat = table.reshape(-1)
    tab2 = jnp.concatenate([tflat, tflat]).reshape(rows_per_pair, lanes)
    flat_in = inputs.reshape(B, D)
    inp2 = jnp.concatenate([flat_in, flat_in], axis=-1)
    inpr = inputs.reshape(B // 2, lanes)

    out2d = pl.pallas_call(
        functools.partial(_body, nsteps, rows_per_pair, half),
        grid=(nsteps,),
        in_specs=[
            pl.BlockSpec(memory_space=pltpu.VMEM),
            pl.BlockSpec(memory_space=pltpu.VMEM),
            pl.BlockSpec(memory_space=pltpu.VMEM),
        ],
        out_specs=pl.BlockSpec(memory_space=pl.ANY),
        out_shape=jax.ShapeDtypeStruct((out_rows, lanes), jnp.float32),
        scratch_shapes=[
            pltpu.VMEM((_NBUF, rows, lanes), jnp.float32),
            pltpu.SemaphoreType.DMA((_NBUF,)),
        ],
    )(inp2, inpr, tab2)
    return out2d.reshape(B, S1, D)


# speedup vs baseline: 7.9121x; 7.9121x over previous
import functools

import jax
import jax.numpy as jnp
from jax.experimental import pallas as pl
from jax.experimental.pallas import tpu as pltpu

_PB = 128
_NBUF = 8


def _body(nsteps, d, b, inT_ref, tlast_ref, tabT_ref, out_ref, scratch, sems):
    i = pl.program_id(0)

    def slab_copy(p, slot):
        return pltpu.make_async_copy(
            scratch.at[slot], out_ref.at[pl.ds(p * d, d)], sems.at[slot])

    for q in range(_PB):
        slot = q % _NBUF
        p = i * _PB + q
        if q < _NBUF:
            @pl.when(i > 0)
            def _wait_prev():
                slab_copy(p - _NBUF, slot).wait()
        else:
            slab_copy(p - _NBUF, slot).wait()
        scratch[slot] = inT_ref[...] + tabT_ref[:, q:q + 1]
        slab_copy(p, slot).start()

    @pl.when(i == nsteps - 1)
    def _tail():
        p_last = nsteps * _PB
        scratch[_NBUF] = inT_ref[...] + tlast_ref[...]
        pltpu.make_async_copy(
            scratch.at[_NBUF], out_ref.at[pl.ds(p_last * d, d)],
            sems.at[_NBUF]).start()
        pltpu.make_async_copy(
            scratch.at[_NBUF], out_ref.at[pl.ds(p_last * d, d)],
            sems.at[_NBUF]).wait()
        for s in range(_NBUF):
            q_last = _PB - _NBUF + s
            slab_copy((nsteps - 1) * _PB + q_last, s).wait()


def kernel(inputs, table):
    B, _, D = inputs.shape
    S1 = table.shape[0]
    nsteps = (S1 - 1) // _PB
    assert nsteps * _PB == S1 - 1

    inT = inputs.reshape(B, D).T
    tabT = table.T
    tlastT = tabT[:, S1 - 1:S1]

    out2d = pl.pallas_call(
        functools.partial(_body, nsteps, D, B),
        grid=(nsteps,),
        in_specs=[
            pl.BlockSpec(memory_space=pltpu.VMEM),
            pl.BlockSpec(memory_space=pltpu.VMEM),
            pl.BlockSpec((D, _PB), lambda i: (0, i)),
        ],
        out_specs=pl.BlockSpec(memory_space=pl.ANY),
        out_shape=jax.ShapeDtypeStruct((S1 * D, B), jnp.float32),
        scratch_shapes=[
            pltpu.VMEM((_NBUF + 1, D, B), jnp.float32),
            pltpu.SemaphoreType.DMA((_NBUF + 1,)),
        ],
    )(inT, tlastT, tabT)
    return out2d.reshape(S1, D, B).transpose(2, 0, 1)
